# Initial kernel scaffold; baseline (speedup 1.0000x reference)
import jax
import jax.numpy as jnp
from jax.experimental import pallas as pl


def kernel(h_v, h_w, e_vw, W):
    raise NotImplementedError("write your pallas kernel here")


# speedup vs baseline: 6.9540x; 6.9540x over previous
import functools

import jax
import jax.numpy as jnp
from jax import lax
from jax.experimental import pallas as pl
from jax.experimental.pallas import tpu as pltpu
from jax.experimental.pallas import tpu_sc as plsc

E = 50000
N_LABELS = 16
OUT = 64

NC = 2
NS = 16
NW = NC * NS
L = 16

PER_W = ((E + NW - 1) // NW + L - 1) // L * L
E_PAD = NW * PER_W
GROUPS = PER_W // L


def _rowsum_body(w_ref, s_ref):
    s_ref[...] = jnp.sum(w_ref[...], axis=-1)


def _rowsum(W):
    return pl.pallas_call(
        _rowsum_body,
        out_shape=jax.ShapeDtypeStruct((N_LABELS, OUT), jnp.float32),
    )(W)


@functools.partial(
    pl.kernel,
    out_type=jax.ShapeDtypeStruct((E_PAD * OUT,), jnp.float32),
    mesh=plsc.VectorSubcoreMesh(core_axis_name="c", subcore_axis_name="s"),
    compiler_params=pltpu.CompilerParams(needs_layout_passes=False),
    scratch_types=[
        pltpu.VMEM((PER_W,), jnp.int32),
        pltpu.VMEM((PER_W,), jnp.float32),
        pltpu.VMEM((N_LABELS * OUT,), jnp.float32),
        pltpu.VMEM((PER_W * OUT,), jnp.float32),
    ],
)
def _sc_lookup(e_hbm, hw_hbm, s_hbm, out_hbm, e_v, hw_v, s_v, out_v):
    wid = lax.axis_index("s") * NC + lax.axis_index("c")
    base = wid * PER_W
    pltpu.sync_copy(e_hbm.at[pl.ds(base, PER_W)], e_v)
    pltpu.sync_copy(hw_hbm.at[pl.ds(base, PER_W)], hw_v)
    pltpu.sync_copy(s_hbm, s_v)

    lane = lax.iota(jnp.int32, L)

    def body(g, carry):
        e16 = e_v[pl.ds(g * L, L)]
        lblbase = jnp.maximum(e16 - 1, 0) * OUT
        hw16 = hw_v[pl.ds(g * L, L)]
        rowbase = (g * L + lane) * OUT
        for c in range(OUT):
            vals = plsc.load_gather(s_v, [lblbase + c])
            plsc.store_scatter(out_v, [rowbase + c], vals * hw16)
        return carry

    lax.fori_loop(0, GROUPS, body, 0)
    pltpu.sync_copy(out_v, out_hbm.at[pl.ds(base * OUT, PER_W * OUT)])


def kernel(h_v, h_w, e_vw, W):
    del h_v
    S = _rowsum(W)
    e_flat = jnp.reshape(e_vw, (-1,)).astype(jnp.int32)
    hw_flat = jnp.reshape(h_w, (-1,)).astype(jnp.float32)
    pad = E_PAD - e_flat.shape[0]
    e_flat = jnp.pad(e_flat, (0, pad))
    hw_flat = jnp.pad(hw_flat, (0, pad))
    out = _sc_lookup(e_flat, hw_flat, jnp.reshape(S, (-1,)))
    return jnp.reshape(out, (E_PAD, OUT))[:E]


# speedup vs baseline: 9.9940x; 1.4372x over previous
import functools

import jax
import jax.numpy as jnp
from jax import lax
from jax.experimental import pallas as pl
from jax.experimental.pallas import tpu as pltpu
from jax.experimental.pallas import tpu_sc as plsc

E = 50000
N_LABELS = 16
OUT = 64

NC = 2
NS = 16
NW = NC * NS
L = 16

PER_W = ((E + NW - 1) // NW + L - 1) // L * L
E_PAD = NW * PER_W
GROUPS = PER_W // L


def _rowsum_body(w_ref, s_ref):
    s_ref[...] = jnp.sum(w_ref[...], axis=-1)


def _rowsum(W):
    return pl.pallas_call(
        _rowsum_body,
        out_shape=jax.ShapeDtypeStruct((N_LABELS, OUT), jnp.float32),
    )(W)


@functools.partial(
    pl.kernel,
    out_type=jax.ShapeDtypeStruct((E_PAD * OUT,), jnp.float32),
    mesh=plsc.VectorSubcoreMesh(core_axis_name="c", subcore_axis_name="s"),
    compiler_params=pltpu.CompilerParams(needs_layout_passes=False),
    scratch_types=[
        pltpu.VMEM((PER_W,), jnp.int32),
        pltpu.VMEM((PER_W,), jnp.float32),
        pltpu.VMEM((N_LABELS * OUT,), jnp.float32),
        pltpu.VMEM((PER_W * OUT,), jnp.float32),
    ],
)
def _sc_lookup(e_hbm, hw_hbm, s_hbm, out_hbm, e_v, hw_v, s_v, out_v):
    wid = lax.axis_index("s") * NC + lax.axis_index("c")
    base = wid * PER_W
    pltpu.sync_copy(e_hbm.at[pl.ds(base, PER_W)], e_v)
    pltpu.sync_copy(hw_hbm.at[pl.ds(base, PER_W)], hw_v)
    pltpu.sync_copy(s_hbm, s_v)

    lane = lax.iota(jnp.int32, L)

    @plsc.parallel_loop(0, GROUPS, unroll=2)
    def _group(g):
        e16 = e_v[pl.ds(g * L, L)]
        lblbase = jnp.maximum(e16 - 1, 0) * OUT
        hw16 = hw_v[pl.ds(g * L, L)]
        rowbase = (g * L + lane) * OUT
        for c in range(OUT):
            vals = plsc.load_gather(s_v, [lblbase + c])
            plsc.store_scatter(out_v, [rowbase + c], vals * hw16)
    pltpu.sync_copy(out_v, out_hbm.at[pl.ds(base * OUT, PER_W * OUT)])


def kernel(h_v, h_w, e_vw, W):
    del h_v
    S = _rowsum(W)
    e_flat = jnp.reshape(e_vw, (-1,)).astype(jnp.int32)
    hw_flat = jnp.reshape(h_w, (-1,)).astype(jnp.float32)
    pad = E_PAD - e_flat.shape[0]
    e_flat = jnp.pad(e_flat, (0, pad))
    hw_flat = jnp.pad(hw_flat, (0, pad))
    out = _sc_lookup(e_flat, hw_flat, jnp.reshape(S, (-1,)))
    return jnp.reshape(out, (E_PAD, OUT))[:E]
